# baseline (speedup 1.0000x reference)
import jax
import jax.numpy as jnp
from jax.experimental import pallas as pl
from jax.experimental.pallas import tpu as pltpu


def _lstm_dir(x, W_ih, W_hh):
    B = x.shape[0]
    H = W_hh.shape[1]
    def step(carry, xt):
        h, c = carry
        gates = xt @ W_ih.T + h @ W_hh.T
        i, f, g, o = jnp.split(gates, 4, axis=1)
        i = jax.nn.sigmoid(i)
        f = jax.nn.sigmoid(f)
        g = jnp.tanh(g)
        o = jax.nn.sigmoid(o)
        c = f * c + i * g
        h = o * jnp.tanh(c)
        return (h, c), None
    init = (jnp.zeros((B, H), x.dtype), jnp.zeros((B, H), x.dtype))
    (h, c), _ = jax.lax.scan(step, init, jnp.swapaxes(x, 0, 1))
    return h


def _bilstm_final(x, W_ih_f, W_hh_f, W_ih_b, W_hh_b):
    h_f = _lstm_dir(x, W_ih_f, W_hh_f)
    h_b = _lstm_dir(x[:, ::-1, :], W_ih_b, W_hh_b)
    return jnp.concatenate([h_f, h_b], axis=1)


def _gcn_conv(x, edge_index, edge_weight, W, b):
    n = x.shape[0]
    src = edge_index[0]
    dst = edge_index[1]
    sl = jnp.arange(n, dtype=src.dtype)
    src2 = jnp.concatenate([src, sl])
    dst2 = jnp.concatenate([dst, sl])
    ew = jnp.concatenate([edge_weight, jnp.ones((n,), x.dtype)])
    deg = jnp.zeros((n,), x.dtype).at[dst2].add(ew)
    dinv = jnp.where(deg > 0, jax.lax.rsqrt(jnp.where(deg > 0, deg, 1.0)), 0.0)
    norm = dinv[src2] * ew * dinv[dst2]
    h = x @ W
    msg = h[src2] * norm[:, None]
    out = jnp.zeros((n, W.shape[1]), x.dtype).at[dst2].add(msg)
    return out + b


def _head_body(patient_ref, family_ref, hgt_ref, pre_W_ref, pre_b_ref,
               com_W_ref, com_b_ref, f1_W_ref, f1_b_ref, f2_W_ref, f2_b_ref,
               lf_W_ref, lf_b_ref, out_ref, p_ref, fam_ref, lstm_ref):
    patient = patient_ref[...]
    family = family_ref[...]
    cat = jnp.concatenate([patient, family], axis=1)
    o = jax.nn.relu(cat @ pre_W_ref[...] + pre_b_ref[...])
    out_ref[...] = jax.nn.sigmoid(o @ com_W_ref[...] + com_b_ref[...])
    p_ref[...] = jax.nn.sigmoid(patient @ f1_W_ref[...] + f1_b_ref[...])
    fam_ref[...] = jax.nn.sigmoid(family @ f2_W_ref[...] + f2_b_ref[...])
    lstm_ref[...] = jax.nn.sigmoid(hgt_ref[...] @ lf_W_ref[...] + lf_b_ref[...])


def _heads(patient, family, hg_t, pre_W, pre_b, com_W, com_b,
           f1_W, f1_b, f2_W, f2_b, lf_W, lf_b):
    B = patient.shape[0]
    BLK = 1024
    grid = (B // BLK,)
    row = lambda i: (i, 0)
    rep = lambda i: (0, 0)
    out_shape = [jax.ShapeDtypeStruct((B, 1), jnp.float32)] * 4
    return pl.pallas_call(
        _head_body,
        grid=grid,
        in_specs=[
            pl.BlockSpec((BLK, 64), row),
            pl.BlockSpec((BLK, 64), row),
            pl.BlockSpec((BLK, 64), row),
            pl.BlockSpec((128, 64), rep),
            pl.BlockSpec((64,), lambda i: (0,)),
            pl.BlockSpec((64, 1), rep),
            pl.BlockSpec((1,), lambda i: (0,)),
            pl.BlockSpec((64, 1), rep),
            pl.BlockSpec((1,), lambda i: (0,)),
            pl.BlockSpec((64, 1), rep),
            pl.BlockSpec((1,), lambda i: (0,)),
            pl.BlockSpec((64, 1), rep),
            pl.BlockSpec((1,), lambda i: (0,)),
        ],
        out_specs=[pl.BlockSpec((BLK, 1), row)] * 4,
        out_shape=out_shape,
    )(patient, family, hg_t, pre_W, pre_b, com_W, com_b,
      f1_W, f1_b, f2_W, f2_b, lf_W, lf_b)


def kernel(x_static_node, x_static_graph, x_longitudinal_node, x_longitudinal_graph, edge_index, edge_weight, batch, target_index, W_ih_f, W_hh_f, W_ih_b, W_hh_b, lin1_W, lin1_b, lin2_W, lin2_b, gcn1_W, gcn1_b, gcn2_W, gcn2_b, pre_W, pre_b, com_W, com_b, f1_W, f1_b, f2_W, f2_b, lstm_fin_W, lstm_fin_b):
    relu = jax.nn.relu
    hp = relu(_bilstm_final(x_longitudinal_node, W_ih_f, W_hh_f, W_ih_b, W_hh_b))
    xc = jnp.concatenate([x_static_node, hp], axis=1)
    patient = relu(xc @ lin1_W + lin1_b)
    patient = relu(patient @ lin2_W + lin2_b)
    hg = relu(_bilstm_final(x_longitudinal_graph, W_ih_f, W_hh_f, W_ih_b, W_hh_b))
    xg = jnp.concatenate([x_static_graph, hg], axis=1)
    g = relu(_gcn_conv(xg, edge_index, edge_weight, gcn1_W, gcn1_b))
    g = relu(_gcn_conv(g, edge_index, edge_weight, gcn2_W, gcn2_b))
    family = g[target_index]
    hg_t = hg[target_index]
    out, p_out, fam_out, lstm_out = _heads(
        patient, family, hg_t, pre_W, pre_b, com_W, com_b,
        f1_W, f1_b, f2_W, f2_b, lstm_fin_W, lstm_fin_b)
    return (out, p_out, fam_out, lstm_out)


# speedup vs baseline: 5.8524x; 5.8524x over previous
import functools

import jax
import jax.numpy as jnp
from jax import lax
from jax.experimental import pallas as pl
from jax.experimental.pallas import tpu as pltpu
from jax.experimental.pallas import tpu_sc as plsc

_NC = 2
_NS = 16
_N = 50000
_E = 800000


def _sc_conv(hs_both, src, dst, ew):
    EPT = _E // _NS
    C = 400
    NCHUNK = EPT // C
    NPAD = 50048
    RPT = NPAD // _NS
    ZR = 136
    mesh = plsc.VectorSubcoreMesh(core_axis_name="c", subcore_axis_name="s")

    @functools.partial(
        pl.kernel,
        out_type=jax.ShapeDtypeStruct((_NC, NPAD, 32), jnp.float32),
        mesh=mesh,
        scratch_types=[
            pltpu.VMEM((C,), jnp.int32),
            pltpu.VMEM((C,), jnp.int32),
            pltpu.VMEM((C,), jnp.float32),
            pltpu.VMEM((C, 32), jnp.float32),
            pltpu.VMEM((ZR, 32), jnp.float32),
            pltpu.VMEM_SHARED((NPAD, 32), jnp.float32),
            pltpu.SemaphoreType.DMA,
        ],
        compiler_params=pltpu.CompilerParams(use_tc_tiling_on_sc=False),
    )
    def k(hs_hbm, src_hbm, dst_hbm, ew_hbm, out_hbm, sidx, didx, ewv, rows, zbuf, acc, sem):
        cid = lax.axis_index("c")
        sid = lax.axis_index("s")
        zv = jnp.zeros((16,), jnp.float32)

        def zrow(i, _):
            zbuf[i, pl.ds(0, 16)] = zv
            zbuf[i, pl.ds(16, 16)] = zv
            return 0
        lax.fori_loop(0, ZR, zrow, 0)

        def zacc(kk, _):
            pltpu.sync_copy(zbuf, acc.at[pl.ds(sid * RPT + kk * ZR, ZR), :])
            return 0
        lax.fori_loop(0, RPT // ZR, zacc, 0)
        plsc.subcore_barrier()

        off = cid * _N

        def chunk(g, _):
            base = sid * EPT + g * C
            pltpu.sync_copy(src_hbm.at[pl.ds(base, C)], sidx)
            pltpu.sync_copy(dst_hbm.at[pl.ds(base, C)], didx)
            pltpu.sync_copy(ew_hbm.at[pl.ds(base, C)], ewv)

            def adj(i, _):
                s = pl.ds(i * 16, 16)
                sidx[s] = sidx[s] + off
                return 0
            lax.fori_loop(0, C // 16, adj, 0)

            pltpu.async_copy(hs_hbm.at[sidx], rows, sem).wait()

            def scale(j, _):
                ev = ewv[pl.ds(j * 16, 16)]
                for l in range(16):
                    lane = jnp.full((16,), l, jnp.int32)
                    e = ev.at[lane].get(mode="promise_in_bounds")
                    r = j * 16 + l
                    rows[r, pl.ds(0, 16)] = rows[r, pl.ds(0, 16)] * e
                    rows[r, pl.ds(16, 16)] = rows[r, pl.ds(16, 16)] * e
                return 0
            lax.fori_loop(0, C // 16, scale, 0)

            pltpu.sync_copy(rows, acc.at[didx], add=True)
            return 0
        lax.fori_loop(0, NCHUNK, chunk, 0)

        plsc.subcore_barrier()
        pltpu.sync_copy(acc.at[pl.ds(sid * RPT, RPT), :],
                        out_hbm.at[cid, pl.ds(sid * RPT, RPT), :])

    return k(hs_both, src, dst, ew)


def _gcn_conv_sc(x_or_h, edge_index, edge_weight, W, b, dinv):
    src = edge_index[0]
    dst = edge_index[1]
    h = x_or_h @ W
    hs = h * dinv[:, None]
    hs_both = jnp.concatenate([hs[:, :32], hs[:, 32:]], axis=0)
    A = _sc_conv(hs_both, src, dst, edge_weight)
    Afull = jnp.concatenate([A[0, :_N], A[1, :_N]], axis=1)
    return dinv[:, None] * (Afull + hs) + b


def _lstm_dir(x, W_ih, W_hh):
    B = x.shape[0]
    H = W_hh.shape[1]
    def step(carry, xt):
        h, c = carry
        gates = xt @ W_ih.T + h @ W_hh.T
        i, f, g, o = jnp.split(gates, 4, axis=1)
        i = jax.nn.sigmoid(i)
        f = jax.nn.sigmoid(f)
        g = jnp.tanh(g)
        o = jax.nn.sigmoid(o)
        c = f * c + i * g
        h = o * jnp.tanh(c)
        return (h, c), None
    init = (jnp.zeros((B, H), x.dtype), jnp.zeros((B, H), x.dtype))
    (h, c), _ = jax.lax.scan(step, init, jnp.swapaxes(x, 0, 1))
    return h


def _bilstm_final(x, W_ih_f, W_hh_f, W_ih_b, W_hh_b):
    h_f = _lstm_dir(x, W_ih_f, W_hh_f)
    h_b = _lstm_dir(x[:, ::-1, :], W_ih_b, W_hh_b)
    return jnp.concatenate([h_f, h_b], axis=1)


def _gcn_conv(x, edge_index, edge_weight, W, b):
    n = x.shape[0]
    src = edge_index[0]
    dst = edge_index[1]
    sl = jnp.arange(n, dtype=src.dtype)
    src2 = jnp.concatenate([src, sl])
    dst2 = jnp.concatenate([dst, sl])
    ew = jnp.concatenate([edge_weight, jnp.ones((n,), x.dtype)])
    deg = jnp.zeros((n,), x.dtype).at[dst2].add(ew)
    dinv = jnp.where(deg > 0, jax.lax.rsqrt(jnp.where(deg > 0, deg, 1.0)), 0.0)
    norm = dinv[src2] * ew * dinv[dst2]
    h = x @ W
    msg = h[src2] * norm[:, None]
    out = jnp.zeros((n, W.shape[1]), x.dtype).at[dst2].add(msg)
    return out + b


def _head_body(patient_ref, family_ref, hgt_ref, pre_W_ref, pre_b_ref,
               com_W_ref, com_b_ref, f1_W_ref, f1_b_ref, f2_W_ref, f2_b_ref,
               lf_W_ref, lf_b_ref, out_ref, p_ref, fam_ref, lstm_ref):
    patient = patient_ref[...]
    family = family_ref[...]
    cat = jnp.concatenate([patient, family], axis=1)
    o = jax.nn.relu(cat @ pre_W_ref[...] + pre_b_ref[...])
    out_ref[...] = jax.nn.sigmoid(o @ com_W_ref[...] + com_b_ref[...])
    p_ref[...] = jax.nn.sigmoid(patient @ f1_W_ref[...] + f1_b_ref[...])
    fam_ref[...] = jax.nn.sigmoid(family @ f2_W_ref[...] + f2_b_ref[...])
    lstm_ref[...] = jax.nn.sigmoid(hgt_ref[...] @ lf_W_ref[...] + lf_b_ref[...])


def _heads(patient, family, hg_t, pre_W, pre_b, com_W, com_b,
           f1_W, f1_b, f2_W, f2_b, lf_W, lf_b):
    B = patient.shape[0]
    BLK = 1024
    grid = (B // BLK,)
    row = lambda i: (i, 0)
    rep = lambda i: (0, 0)
    out_shape = [jax.ShapeDtypeStruct((B, 1), jnp.float32)] * 4
    return pl.pallas_call(
        _head_body,
        grid=grid,
        in_specs=[
            pl.BlockSpec((BLK, 64), row),
            pl.BlockSpec((BLK, 64), row),
            pl.BlockSpec((BLK, 64), row),
            pl.BlockSpec((128, 64), rep),
            pl.BlockSpec((64,), lambda i: (0,)),
            pl.BlockSpec((64, 1), rep),
            pl.BlockSpec((1,), lambda i: (0,)),
            pl.BlockSpec((64, 1), rep),
            pl.BlockSpec((1,), lambda i: (0,)),
            pl.BlockSpec((64, 1), rep),
            pl.BlockSpec((1,), lambda i: (0,)),
            pl.BlockSpec((64, 1), rep),
            pl.BlockSpec((1,), lambda i: (0,)),
        ],
        out_specs=[pl.BlockSpec((BLK, 1), row)] * 4,
        out_shape=out_shape,
    )(patient, family, hg_t, pre_W, pre_b, com_W, com_b,
      f1_W, f1_b, f2_W, f2_b, lf_W, lf_b)


def kernel(x_static_node, x_static_graph, x_longitudinal_node, x_longitudinal_graph, edge_index, edge_weight, batch, target_index, W_ih_f, W_hh_f, W_ih_b, W_hh_b, lin1_W, lin1_b, lin2_W, lin2_b, gcn1_W, gcn1_b, gcn2_W, gcn2_b, pre_W, pre_b, com_W, com_b, f1_W, f1_b, f2_W, f2_b, lstm_fin_W, lstm_fin_b):
    relu = jax.nn.relu
    hp = relu(_bilstm_final(x_longitudinal_node, W_ih_f, W_hh_f, W_ih_b, W_hh_b))
    xc = jnp.concatenate([x_static_node, hp], axis=1)
    patient = relu(xc @ lin1_W + lin1_b)
    patient = relu(patient @ lin2_W + lin2_b)
    hg = relu(_bilstm_final(x_longitudinal_graph, W_ih_f, W_hh_f, W_ih_b, W_hh_b))
    xg = jnp.concatenate([x_static_graph, hg], axis=1)
    n = xg.shape[0]
    deg = jnp.ones((n,), jnp.float32).at[edge_index[1]].add(edge_weight)
    dinv = jax.lax.rsqrt(deg)
    g = relu(_gcn_conv_sc(xg, edge_index, edge_weight, gcn1_W, gcn1_b, dinv))
    g = relu(_gcn_conv_sc(g, edge_index, edge_weight, gcn2_W, gcn2_b, dinv))
    family = g[target_index]
    hg_t = hg[target_index]
    out, p_out, fam_out, lstm_out = _heads(
        patient, family, hg_t, pre_W, pre_b, com_W, com_b,
        f1_W, f1_b, f2_W, f2_b, lstm_fin_W, lstm_fin_b)
    return (out, p_out, fam_out, lstm_out)


# speedup vs baseline: 8.5202x; 1.4559x over previous
import functools

import jax
import jax.numpy as jnp
from jax import lax
from jax.experimental import pallas as pl
from jax.experimental.pallas import tpu as pltpu
from jax.experimental.pallas import tpu_sc as plsc

_NC = 2
_NS = 16
_N = 50000
_E = 800000


def _sc_conv(hs_both, src, dst, ew):
    EPT = _E // _NS
    C = 400
    NCHUNK = EPT // C
    NPAD = 50048
    RPT = NPAD // _NS
    ZR = 136
    mesh = plsc.VectorSubcoreMesh(core_axis_name="c", subcore_axis_name="s")

    @functools.partial(
        pl.kernel,
        out_type=jax.ShapeDtypeStruct((_NC, NPAD, 32), jnp.float32),
        mesh=mesh,
        scratch_types=[
            pltpu.VMEM((C,), jnp.int32),
            pltpu.VMEM((C,), jnp.int32),
            pltpu.VMEM((C,), jnp.float32),
            pltpu.VMEM((C, 32), jnp.float32),
            pltpu.VMEM((ZR, 32), jnp.float32),
            pltpu.VMEM_SHARED((NPAD, 32), jnp.float32),
            pltpu.SemaphoreType.DMA,
        ],
        compiler_params=pltpu.CompilerParams(use_tc_tiling_on_sc=False),
    )
    def k(hs_hbm, src_hbm, dst_hbm, ew_hbm, out_hbm, sidx, didx, ewv, rows, zbuf, acc, sem):
        cid = lax.axis_index("c")
        sid = lax.axis_index("s")
        zv = jnp.zeros((16,), jnp.float32)

        def zrow(i, _):
            zbuf[i, pl.ds(0, 16)] = zv
            zbuf[i, pl.ds(16, 16)] = zv
            return 0
        lax.fori_loop(0, ZR, zrow, 0)

        def zacc(kk, _):
            pltpu.sync_copy(zbuf, acc.at[pl.ds(sid * RPT + kk * ZR, ZR), :])
            return 0
        lax.fori_loop(0, RPT // ZR, zacc, 0)
        plsc.subcore_barrier()

        off = cid * _N

        def chunk(g, _):
            base = sid * EPT + g * C
            pltpu.sync_copy(src_hbm.at[pl.ds(base, C)], sidx)
            pltpu.sync_copy(dst_hbm.at[pl.ds(base, C)], didx)
            pltpu.sync_copy(ew_hbm.at[pl.ds(base, C)], ewv)

            def adj(i, _):
                s = pl.ds(i * 16, 16)
                sidx[s] = sidx[s] + off
                return 0
            lax.fori_loop(0, C // 16, adj, 0)

            pltpu.async_copy(hs_hbm.at[sidx], rows, sem).wait()

            def scale(j, _):
                ev = ewv[pl.ds(j * 16, 16)]
                for l in range(16):
                    lane = jnp.full((16,), l, jnp.int32)
                    e = ev.at[lane].get(mode="promise_in_bounds")
                    r = j * 16 + l
                    rows[r, pl.ds(0, 16)] = rows[r, pl.ds(0, 16)] * e
                    rows[r, pl.ds(16, 16)] = rows[r, pl.ds(16, 16)] * e
                return 0
            lax.fori_loop(0, C // 16, scale, 0)

            pltpu.sync_copy(rows, acc.at[didx], add=True)
            return 0
        lax.fori_loop(0, NCHUNK, chunk, 0)

        plsc.subcore_barrier()
        pltpu.sync_copy(acc.at[pl.ds(sid * RPT, RPT), :],
                        out_hbm.at[cid, pl.ds(sid * RPT, RPT), :])

    return k(hs_both, src, dst, ew)


def _gcn_conv_sc(x_or_h, edge_index, edge_weight, W, b, dinv):
    src = edge_index[0]
    dst = edge_index[1]
    h = x_or_h @ W
    hs = h * dinv[:, None]
    hs_both = jnp.concatenate([hs[:, :32], hs[:, 32:]], axis=0)
    A = _sc_conv(hs_both, src, dst, edge_weight)
    Afull = jnp.concatenate([A[0, :_N], A[1, :_N]], axis=1)
    return dinv[:, None] * (Afull + hs) + b


def _bilstm_tc(xT, WfT, WbT):
    T = xT.shape[0]
    NP = xT.shape[2]
    R = 512
    sig = jax.nn.sigmoid

    def body(xT_ref, wf_ref, wb_ref, out_ref):
        wf = wf_ref[...]
        wb = wb_ref[...]

        def step(t, carry):
            hf, cf, hb, cb = carry
            xtf = xT_ref[t]
            xtb = xT_ref[T - 1 - t]
            gf = jax.lax.dot_general(
                wf, jnp.concatenate([xtf, hf], axis=0),
                (((1,), (0,)), ((), ())), preferred_element_type=jnp.float32)
            gb = jax.lax.dot_general(
                wb, jnp.concatenate([xtb, hb], axis=0),
                (((1,), (0,)), ((), ())), preferred_element_type=jnp.float32)
            cf = sig(gf[32:64]) * cf + sig(gf[0:32]) * jnp.tanh(gf[64:96])
            hf = sig(gf[96:128]) * jnp.tanh(cf)
            cb = sig(gb[32:64]) * cb + sig(gb[0:32]) * jnp.tanh(gb[64:96])
            hb = sig(gb[96:128]) * jnp.tanh(cb)
            return (hf, cf, hb, cb)

        z = jnp.zeros((32, R), jnp.float32)
        hf, cf, hb, cb = lax.fori_loop(0, T, step, (z, z, z, z))
        out_ref[0:32, :] = jnp.maximum(hf, 0.0)
        out_ref[32:64, :] = jnp.maximum(hb, 0.0)

    return pl.pallas_call(
        body,
        grid=(NP // R,),
        in_specs=[
            pl.BlockSpec((T, 16, R), lambda i: (0, 0, i)),
            pl.BlockSpec((128, 48), lambda i: (0, 0)),
            pl.BlockSpec((128, 48), lambda i: (0, 0)),
        ],
        out_specs=pl.BlockSpec((64, R), lambda i: (0, i)),
        out_shape=jax.ShapeDtypeStruct((64, NP), jnp.float32),
    )(xT, WfT, WbT)


def _lstm_dir(x, W_ih, W_hh):
    B = x.shape[0]
    H = W_hh.shape[1]
    def step(carry, xt):
        h, c = carry
        gates = xt @ W_ih.T + h @ W_hh.T
        i, f, g, o = jnp.split(gates, 4, axis=1)
        i = jax.nn.sigmoid(i)
        f = jax.nn.sigmoid(f)
        g = jnp.tanh(g)
        o = jax.nn.sigmoid(o)
        c = f * c + i * g
        h = o * jnp.tanh(c)
        return (h, c), None
    init = (jnp.zeros((B, H), x.dtype), jnp.zeros((B, H), x.dtype))
    (h, c), _ = jax.lax.scan(step, init, jnp.swapaxes(x, 0, 1))
    return h


def _bilstm_final(x, W_ih_f, W_hh_f, W_ih_b, W_hh_b):
    h_f = _lstm_dir(x, W_ih_f, W_hh_f)
    h_b = _lstm_dir(x[:, ::-1, :], W_ih_b, W_hh_b)
    return jnp.concatenate([h_f, h_b], axis=1)


def _gcn_conv(x, edge_index, edge_weight, W, b):
    n = x.shape[0]
    src = edge_index[0]
    dst = edge_index[1]
    sl = jnp.arange(n, dtype=src.dtype)
    src2 = jnp.concatenate([src, sl])
    dst2 = jnp.concatenate([dst, sl])
    ew = jnp.concatenate([edge_weight, jnp.ones((n,), x.dtype)])
    deg = jnp.zeros((n,), x.dtype).at[dst2].add(ew)
    dinv = jnp.where(deg > 0, jax.lax.rsqrt(jnp.where(deg > 0, deg, 1.0)), 0.0)
    norm = dinv[src2] * ew * dinv[dst2]
    h = x @ W
    msg = h[src2] * norm[:, None]
    out = jnp.zeros((n, W.shape[1]), x.dtype).at[dst2].add(msg)
    return out + b


def _head_body(patient_ref, family_ref, hgt_ref, pre_W_ref, pre_b_ref,
               com_W_ref, com_b_ref, f1_W_ref, f1_b_ref, f2_W_ref, f2_b_ref,
               lf_W_ref, lf_b_ref, out_ref, p_ref, fam_ref, lstm_ref):
    patient = patient_ref[...]
    family = family_ref[...]
    cat = jnp.concatenate([patient, family], axis=1)
    o = jax.nn.relu(cat @ pre_W_ref[...] + pre_b_ref[...])
    out_ref[...] = jax.nn.sigmoid(o @ com_W_ref[...] + com_b_ref[...])
    p_ref[...] = jax.nn.sigmoid(patient @ f1_W_ref[...] + f1_b_ref[...])
    fam_ref[...] = jax.nn.sigmoid(family @ f2_W_ref[...] + f2_b_ref[...])
    lstm_ref[...] = jax.nn.sigmoid(hgt_ref[...] @ lf_W_ref[...] + lf_b_ref[...])


def _heads(patient, family, hg_t, pre_W, pre_b, com_W, com_b,
           f1_W, f1_b, f2_W, f2_b, lf_W, lf_b):
    B = patient.shape[0]
    BLK = 1024
    grid = (B // BLK,)
    row = lambda i: (i, 0)
    rep = lambda i: (0, 0)
    out_shape = [jax.ShapeDtypeStruct((B, 1), jnp.float32)] * 4
    return pl.pallas_call(
        _head_body,
        grid=grid,
        in_specs=[
            pl.BlockSpec((BLK, 64), row),
            pl.BlockSpec((BLK, 64), row),
            pl.BlockSpec((BLK, 64), row),
            pl.BlockSpec((128, 64), rep),
            pl.BlockSpec((64,), lambda i: (0,)),
            pl.BlockSpec((64, 1), rep),
            pl.BlockSpec((1,), lambda i: (0,)),
            pl.BlockSpec((64, 1), rep),
            pl.BlockSpec((1,), lambda i: (0,)),
            pl.BlockSpec((64, 1), rep),
            pl.BlockSpec((1,), lambda i: (0,)),
            pl.BlockSpec((64, 1), rep),
            pl.BlockSpec((1,), lambda i: (0,)),
        ],
        out_specs=[pl.BlockSpec((BLK, 1), row)] * 4,
        out_shape=out_shape,
    )(patient, family, hg_t, pre_W, pre_b, com_W, com_b,
      f1_W, f1_b, f2_W, f2_b, lf_W, lf_b)


def kernel(x_static_node, x_static_graph, x_longitudinal_node, x_longitudinal_graph, edge_index, edge_weight, batch, target_index, W_ih_f, W_hh_f, W_ih_b, W_hh_b, lin1_W, lin1_b, lin2_W, lin2_b, gcn1_W, gcn1_b, gcn2_W, gcn2_b, pre_W, pre_b, com_W, com_b, f1_W, f1_b, f2_W, f2_b, lstm_fin_W, lstm_fin_b):
    relu = jax.nn.relu
    NPK = 54272
    nn = x_longitudinal_graph.shape[0] + x_longitudinal_node.shape[0]
    xcomb = jnp.concatenate([x_longitudinal_graph, x_longitudinal_node], axis=0)
    xT = jnp.pad(jnp.transpose(xcomb, (1, 2, 0)), ((0, 0), (0, 0), (0, NPK - nn)))
    WfT = jnp.concatenate([W_ih_f, W_hh_f], axis=1)
    WbT = jnp.concatenate([W_ih_b, W_hh_b], axis=1)
    hgT = _bilstm_tc(xT, WfT, WbT)
    hg = hgT[:, :50000].T
    hp = hgT[:, 50000:nn].T
    xc = jnp.concatenate([x_static_node, hp], axis=1)
    patient = relu(xc @ lin1_W + lin1_b)
    patient = relu(patient @ lin2_W + lin2_b)
    xg = jnp.concatenate([x_static_graph, hg], axis=1)
    n = xg.shape[0]
    deg = jnp.ones((n,), jnp.float32).at[edge_index[1]].add(edge_weight)
    dinv = jax.lax.rsqrt(deg)
    g = relu(_gcn_conv_sc(xg, edge_index, edge_weight, gcn1_W, gcn1_b, dinv))
    g = relu(_gcn_conv_sc(g, edge_index, edge_weight, gcn2_W, gcn2_b, dinv))
    family = g[target_index]
    hg_t = hg[target_index]
    out, p_out, fam_out, lstm_out = _heads(
        patient, family, hg_t, pre_W, pre_b, com_W, com_b,
        f1_W, f1_b, f2_W, f2_b, lstm_fin_W, lstm_fin_b)
    return (out, p_out, fam_out, lstm_out)


# speedup vs baseline: 15.9748x; 1.8749x over previous
import functools

import jax
import jax.numpy as jnp
from jax import lax
from jax.experimental import pallas as pl
from jax.experimental.pallas import tpu as pltpu
from jax.experimental.pallas import tpu_sc as plsc

_NC = 2
_NS = 16
_N = 50000
_E = 800000


_CCH = 400
_NCHUNK = _E // (_NS * _CCH)
_NPAD = 50048
_GRP = 4


def _sc_conv(hsA, hsB, ei3, ew2):
    C = _CCH
    CPT = _NCHUNK
    NGRP = (CPT - 1) // _GRP
    RPT = _NPAD // _NS
    mesh = plsc.VectorSubcoreMesh(core_axis_name="c", subcore_axis_name="s")

    @functools.partial(
        pl.kernel,
        out_type=jax.ShapeDtypeStruct((_NC, _NPAD, 32), jnp.float32),
        mesh=mesh,
        scratch_types=[
            pltpu.VMEM((2, _GRP, C), jnp.int32),
            pltpu.VMEM((_GRP, C), jnp.float32),
            pltpu.VMEM((C, 32), jnp.float32),
            pltpu.VMEM((C, 32), jnp.float32),
            pltpu.VMEM_SHARED((_NPAD, 32), jnp.float32),
            pltpu.SemaphoreType.DMA,
            pltpu.SemaphoreType.DMA,
            pltpu.SemaphoreType.DMA,
        ],
        compiler_params=pltpu.CompilerParams(use_tc_tiling_on_sc=False),
    )
    def k(hsA_hbm, hsB_hbm, ei_hbm, ew_hbm, out_hbm, eidx, ewg, rows0, rows1,
          acc, sem0, sem1, sems):
        cid = lax.axis_index("c")
        sid = lax.axis_index("s")
        rowsb = (rows0, rows1)
        semb = (sem0, sem1)
        zv = jnp.zeros((16,), jnp.float32)

        def zrow(i, _):
            rows0[i, pl.ds(0, 16)] = zv
            rows0[i, pl.ds(16, 16)] = zv
            return 0
        lax.fori_loop(0, C, zrow, 0)
        for kk in range(RPT // C):
            pltpu.sync_copy(rows0, acc.at[pl.ds(sid * RPT + kk * C, C), :])
        rem = RPT % C
        pltpu.sync_copy(rows0.at[pl.ds(0, rem), :],
                        acc.at[pl.ds(sid * RPT + (RPT // C) * C, rem), :])
        plsc.subcore_barrier()

        def start_gather(j, b):
            sidx = eidx.at[0, j]

            @pl.when(cid == 0)
            def _():
                pltpu.async_copy(hsA_hbm.at[sidx], rowsb[b], semb[b])

            @pl.when(cid == 1)
            def _():
                pltpu.async_copy(hsB_hbm.at[sidx], rowsb[b], semb[b])

        def wait_gather(j, b):
            pltpu.make_async_copy(hsA_hbm.at[eidx.at[0, j]], rowsb[b], semb[b]).wait()

        def scale(j, b):
            rr = rowsb[b]

            def sc16(m, _):
                ev = ewg[j, pl.ds(m * 16, 16)]
                for l in range(16):
                    lane = jnp.full((16,), l, jnp.int32)
                    e = ev.at[lane].get(mode="promise_in_bounds")
                    r = m * 16 + l
                    rr[r, pl.ds(0, 16)] = rr[r, pl.ds(0, 16)] * e
                    rr[r, pl.ds(16, 16)] = rr[r, pl.ds(16, 16)] * e
                return 0
            lax.fori_loop(0, C // 16, sc16, 0)

        def start_scatter(j, b):
            pltpu.async_copy(rowsb[b], acc.at[eidx.at[1, j]], sems, add=True)

        def drain_scatter(j, b):
            pltpu.make_async_copy(rowsb[b], acc.at[eidx.at[1, j]], sems).wait()

        def group(m, _):
            gb = sid * CPT + m * _GRP
            pltpu.sync_copy(ei_hbm.at[:, pl.ds(gb, _GRP), :], eidx)
            pltpu.sync_copy(ew_hbm.at[pl.ds(gb, _GRP), :], ewg)
            start_gather(0, 0)
            start_gather(1, 1)
            wait_gather(0, 0)
            scale(0, 0)
            start_scatter(0, 0)
            wait_gather(1, 1)
            scale(1, 1)
            start_scatter(1, 1)
            drain_scatter(0, 0)
            start_gather(2, 0)
            drain_scatter(1, 1)
            start_gather(3, 1)
            wait_gather(2, 0)
            scale(2, 0)
            start_scatter(2, 0)
            wait_gather(3, 1)
            scale(3, 1)
            start_scatter(3, 1)
            drain_scatter(2, 0)
            drain_scatter(3, 1)
            return 0
        lax.fori_loop(0, NGRP, group, 0)

        gb = sid * CPT + NGRP * _GRP
        pltpu.sync_copy(ei_hbm.at[:, pl.ds(gb, 1), :], eidx.at[:, pl.ds(0, 1), :])
        pltpu.sync_copy(ew_hbm.at[pl.ds(gb, 1), :], ewg.at[pl.ds(0, 1), :])
        start_gather(0, 0)
        wait_gather(0, 0)
        scale(0, 0)
        pltpu.sync_copy(rows0, acc.at[eidx.at[1, 0]], add=True)

        plsc.subcore_barrier()
        pltpu.sync_copy(acc.at[pl.ds(sid * RPT, RPT), :],
                        out_hbm.at[cid, pl.ds(sid * RPT, RPT), :])

    return k(hsA, hsB, ei3, ew2)


def _sc_deg(ei3, ew2):
    C = 2000
    CPT = _E // (_NS * C)
    RPT = _NPAD // _NS
    mesh = plsc.VectorSubcoreMesh(core_axis_name="c", subcore_axis_name="s")

    @functools.partial(
        pl.kernel,
        out_type=jax.ShapeDtypeStruct((_NC, _NPAD, 16), jnp.float32),
        mesh=mesh,
        scratch_types=[
            pltpu.VMEM((C // _CCH, _CCH), jnp.int32),
            pltpu.VMEM((C // _CCH, _CCH), jnp.float32),
            pltpu.VMEM((C, 16), jnp.float32),
            pltpu.VMEM_SHARED((_NPAD, 16), jnp.float32),
        ],
        compiler_params=pltpu.CompilerParams(use_tc_tiling_on_sc=False,
                                             needs_layout_passes=False),
    )
    def k(ei_hbm, ew_hbm, out_hbm, didx, ewv, rows16, acc):
        cid = lax.axis_index("c")
        sid = lax.axis_index("s")
        Q = C // _CCH
        zv = jnp.zeros((16,), jnp.float32)

        def zrow(i, _):
            rows16[i, pl.ds(0, 16)] = zv
            return 0
        lax.fori_loop(0, C, zrow, 0)
        pltpu.sync_copy(rows16.at[pl.ds(0, RPT - C), :],
                        acc.at[pl.ds(sid * RPT, RPT - C), :])
        pltpu.sync_copy(rows16,
                        acc.at[pl.ds(sid * RPT + (RPT - C), C), :])
        plsc.subcore_barrier()

        col0 = jnp.zeros((16,), jnp.int32)
        lanes = lax.iota(jnp.int32, 16)

        def macro(g, _):
            base = sid * (_E // (_NS * _CCH)) + g * Q
            pltpu.sync_copy(ei_hbm.at[1, pl.ds(base, Q), :], didx)
            pltpu.sync_copy(ew_hbm.at[pl.ds(base, Q), :], ewv)
            for q in range(Q):
                def sc16(m, _):
                    ev = ewv[q, pl.ds(m * 16, 16)]
                    ridx = lanes + (q * _CCH + m * 16)
                    plsc.store_scatter(rows16, [ridx, col0], ev)
                    return 0
                lax.fori_loop(0, _CCH // 16, sc16, 0)
            for q in range(Q):
                pltpu.sync_copy(rows16.at[pl.ds(q * _CCH, _CCH), :],
                                acc.at[didx.at[q]], add=True)
            return 0
        lax.fori_loop(0, CPT, macro, 0)

        plsc.subcore_barrier()
        pltpu.sync_copy(acc.at[pl.ds(sid * RPT, RPT), :],
                        out_hbm.at[cid, pl.ds(sid * RPT, RPT), :])

    return k(ei3, ew2)


def _gcn_conv_sc(x_or_h, ei3, ew2, W, b, dinv):
    h = x_or_h @ W
    hs = h * dinv[:, None]
    A = _sc_conv(hs[:, :32], hs[:, 32:], ei3, ew2)
    Afull = jnp.concatenate([A[0, :_N], A[1, :_N]], axis=1)
    return dinv[:, None] * (Afull + hs) + b


def _bilstm_tc(xT, WfT, WbT):
    T = xT.shape[0]
    NP = xT.shape[2]
    R = 512
    sig = jax.nn.sigmoid

    def body(xT_ref, wf_ref, wb_ref, out_ref):
        wf = wf_ref[...]
        wb = wb_ref[...]

        def step(t, carry):
            hf, cf, hb, cb = carry
            xtf = xT_ref[t]
            xtb = xT_ref[T - 1 - t]
            gf = jax.lax.dot_general(
                wf, jnp.concatenate([xtf, hf], axis=0),
                (((1,), (0,)), ((), ())), preferred_element_type=jnp.float32)
            gb = jax.lax.dot_general(
                wb, jnp.concatenate([xtb, hb], axis=0),
                (((1,), (0,)), ((), ())), preferred_element_type=jnp.float32)
            cf = sig(gf[32:64]) * cf + sig(gf[0:32]) * jnp.tanh(gf[64:96])
            hf = sig(gf[96:128]) * jnp.tanh(cf)
            cb = sig(gb[32:64]) * cb + sig(gb[0:32]) * jnp.tanh(gb[64:96])
            hb = sig(gb[96:128]) * jnp.tanh(cb)
            return (hf, cf, hb, cb)

        z = jnp.zeros((32, R), jnp.float32)
        hf, cf, hb, cb = lax.fori_loop(0, T, step, (z, z, z, z))
        out_ref[0:32, :] = jnp.maximum(hf, 0.0)
        out_ref[32:64, :] = jnp.maximum(hb, 0.0)

    return pl.pallas_call(
        body,
        grid=(NP // R,),
        in_specs=[
            pl.BlockSpec((T, 16, R), lambda i: (0, 0, i)),
            pl.BlockSpec((128, 48), lambda i: (0, 0)),
            pl.BlockSpec((128, 48), lambda i: (0, 0)),
        ],
        out_specs=pl.BlockSpec((64, R), lambda i: (0, i)),
        out_shape=jax.ShapeDtypeStruct((64, NP), jnp.float32),
    )(xT, WfT, WbT)


def _lstm_dir(x, W_ih, W_hh):
    B = x.shape[0]
    H = W_hh.shape[1]
    def step(carry, xt):
        h, c = carry
        gates = xt @ W_ih.T + h @ W_hh.T
        i, f, g, o = jnp.split(gates, 4, axis=1)
        i = jax.nn.sigmoid(i)
        f = jax.nn.sigmoid(f)
        g = jnp.tanh(g)
        o = jax.nn.sigmoid(o)
        c = f * c + i * g
        h = o * jnp.tanh(c)
        return (h, c), None
    init = (jnp.zeros((B, H), x.dtype), jnp.zeros((B, H), x.dtype))
    (h, c), _ = jax.lax.scan(step, init, jnp.swapaxes(x, 0, 1))
    return h


def _bilstm_final(x, W_ih_f, W_hh_f, W_ih_b, W_hh_b):
    h_f = _lstm_dir(x, W_ih_f, W_hh_f)
    h_b = _lstm_dir(x[:, ::-1, :], W_ih_b, W_hh_b)
    return jnp.concatenate([h_f, h_b], axis=1)


def _gcn_conv(x, edge_index, edge_weight, W, b):
    n = x.shape[0]
    src = edge_index[0]
    dst = edge_index[1]
    sl = jnp.arange(n, dtype=src.dtype)
    src2 = jnp.concatenate([src, sl])
    dst2 = jnp.concatenate([dst, sl])
    ew = jnp.concatenate([edge_weight, jnp.ones((n,), x.dtype)])
    deg = jnp.zeros((n,), x.dtype).at[dst2].add(ew)
    dinv = jnp.where(deg > 0, jax.lax.rsqrt(jnp.where(deg > 0, deg, 1.0)), 0.0)
    norm = dinv[src2] * ew * dinv[dst2]
    h = x @ W
    msg = h[src2] * norm[:, None]
    out = jnp.zeros((n, W.shape[1]), x.dtype).at[dst2].add(msg)
    return out + b


def _head_body(patient_ref, family_ref, hgt_ref, pre_W_ref, pre_b_ref,
               com_W_ref, com_b_ref, f1_W_ref, f1_b_ref, f2_W_ref, f2_b_ref,
               lf_W_ref, lf_b_ref, out_ref, p_ref, fam_ref, lstm_ref):
    patient = patient_ref[...]
    family = family_ref[...]
    cat = jnp.concatenate([patient, family], axis=1)
    o = jax.nn.relu(cat @ pre_W_ref[...] + pre_b_ref[...])
    out_ref[...] = jax.nn.sigmoid(o @ com_W_ref[...] + com_b_ref[...])
    p_ref[...] = jax.nn.sigmoid(patient @ f1_W_ref[...] + f1_b_ref[...])
    fam_ref[...] = jax.nn.sigmoid(family @ f2_W_ref[...] + f2_b_ref[...])
    lstm_ref[...] = jax.nn.sigmoid(hgt_ref[...] @ lf_W_ref[...] + lf_b_ref[...])


def _heads(patient, family, hg_t, pre_W, pre_b, com_W, com_b,
           f1_W, f1_b, f2_W, f2_b, lf_W, lf_b):
    B = patient.shape[0]
    BLK = 1024
    grid = (B // BLK,)
    row = lambda i: (i, 0)
    rep = lambda i: (0, 0)
    out_shape = [jax.ShapeDtypeStruct((B, 1), jnp.float32)] * 4
    return pl.pallas_call(
        _head_body,
        grid=grid,
        in_specs=[
            pl.BlockSpec((BLK, 64), row),
            pl.BlockSpec((BLK, 64), row),
            pl.BlockSpec((BLK, 64), row),
            pl.BlockSpec((128, 64), rep),
            pl.BlockSpec((64,), lambda i: (0,)),
            pl.BlockSpec((64, 1), rep),
            pl.BlockSpec((1,), lambda i: (0,)),
            pl.BlockSpec((64, 1), rep),
            pl.BlockSpec((1,), lambda i: (0,)),
            pl.BlockSpec((64, 1), rep),
            pl.BlockSpec((1,), lambda i: (0,)),
            pl.BlockSpec((64, 1), rep),
            pl.BlockSpec((1,), lambda i: (0,)),
        ],
        out_specs=[pl.BlockSpec((BLK, 1), row)] * 4,
        out_shape=out_shape,
    )(patient, family, hg_t, pre_W, pre_b, com_W, com_b,
      f1_W, f1_b, f2_W, f2_b, lf_W, lf_b)


def kernel(x_static_node, x_static_graph, x_longitudinal_node, x_longitudinal_graph, edge_index, edge_weight, batch, target_index, W_ih_f, W_hh_f, W_ih_b, W_hh_b, lin1_W, lin1_b, lin2_W, lin2_b, gcn1_W, gcn1_b, gcn2_W, gcn2_b, pre_W, pre_b, com_W, com_b, f1_W, f1_b, f2_W, f2_b, lstm_fin_W, lstm_fin_b):
    relu = jax.nn.relu
    NPK = 54272
    nn = x_longitudinal_graph.shape[0] + x_longitudinal_node.shape[0]
    xcomb = jnp.concatenate([x_longitudinal_graph, x_longitudinal_node], axis=0)
    xT = jnp.pad(jnp.transpose(xcomb, (1, 2, 0)), ((0, 0), (0, 0), (0, NPK - nn)))
    WfT = jnp.concatenate([W_ih_f, W_hh_f], axis=1)
    WbT = jnp.concatenate([W_ih_b, W_hh_b], axis=1)
    hgT = _bilstm_tc(xT, WfT, WbT)
    hg = hgT[:, :50000].T
    hp = hgT[:, 50000:nn].T
    xc = jnp.concatenate([x_static_node, hp], axis=1)
    patient = relu(xc @ lin1_W + lin1_b)
    patient = relu(patient @ lin2_W + lin2_b)
    xg = jnp.concatenate([x_static_graph, hg], axis=1)
    ei3 = edge_index.reshape(2, _E // _CCH, _CCH)
    ew2 = edge_weight.reshape(_E // _CCH, _CCH)
    deg16 = _sc_deg(ei3, ew2)
    dinv = jax.lax.rsqrt(1.0 + deg16[0, :_N, 0])
    g = relu(_gcn_conv_sc(xg, ei3, ew2, gcn1_W, gcn1_b, dinv))
    g = relu(_gcn_conv_sc(g, ei3, ew2, gcn2_W, gcn2_b, dinv))
    family = g[target_index]
    hg_t = hg[target_index]
    out, p_out, fam_out, lstm_out = _heads(
        patient, family, hg_t, pre_W, pre_b, com_W, com_b,
        f1_W, f1_b, f2_W, f2_b, lstm_fin_W, lstm_fin_b)
    return (out, p_out, fam_out, lstm_out)
